# baseline (speedup 1.0000x reference)
import jax
import jax.numpy as jnp
from jax.experimental import pallas as pl
from jax.experimental.pallas import tpu as pltpu

_BM = 400
_BSUP = 2000


def _support_body(x_ref, w1_ref, out_ref):
    out_ref[...] = jnp.dot(
        x_ref[...], w1_ref[...], preferred_element_type=jnp.float32
    )


def _layer1_body(adj_ref, sup_ref, b1_ref, w2_ref, out_ref):
    acc = jnp.dot(
        adj_ref[...], sup_ref[...],
        preferred_element_type=jnp.float32,
    )
    h = jnp.maximum(acc + b1_ref[...], 0.0)
    out_ref[...] = jnp.dot(
        h, w2_ref[...], preferred_element_type=jnp.float32
    )


def _layer2_body(adj_ref, s2_ref, b2_ref, out_ref):
    out_ref[...] = jnp.dot(
        adj_ref[...], s2_ref[...],
        preferred_element_type=jnp.float32,
    ) + b2_ref[...]


def kernel(x, adj, W1, b1, W2, b2):
    n, nfeat = x.shape
    h1 = W1.shape[1]
    h2 = W2.shape[1]
    b1r = b1.reshape(1, h1)
    b2r = b2.reshape(1, h2)

    support = pl.pallas_call(
        _support_body,
        grid=(n // _BSUP,),
        in_specs=[
            pl.BlockSpec((_BSUP, nfeat), lambda i: (i, 0)),
            pl.BlockSpec((nfeat, h1), lambda i: (0, 0)),
        ],
        out_specs=pl.BlockSpec((_BSUP, h1), lambda i: (i, 0)),
        out_shape=jax.ShapeDtypeStruct((n, h1), jnp.float32),
    )(x, W1)

    s2 = pl.pallas_call(
        _layer1_body,
        grid=(n // _BM,),
        in_specs=[
            pl.BlockSpec((_BM, n), lambda i: (i, 0)),
            pl.BlockSpec((n, h1), lambda i: (0, 0)),
            pl.BlockSpec((1, h1), lambda i: (0, 0)),
            pl.BlockSpec((h1, h2), lambda i: (0, 0)),
        ],
        out_specs=pl.BlockSpec((_BM, h2), lambda i: (i, 0)),
        out_shape=jax.ShapeDtypeStruct((n, h2), jnp.float32),
        compiler_params=pltpu.CompilerParams(
            dimension_semantics=("arbitrary",),
        ),
    )(adj, support, b1r, W2)

    out = pl.pallas_call(
        _layer2_body,
        grid=(n // _BM,),
        in_specs=[
            pl.BlockSpec((_BM, n), lambda i: (i, 0)),
            pl.BlockSpec((n, h2), lambda i: (0, 0)),
            pl.BlockSpec((1, h2), lambda i: (0, 0)),
        ],
        out_specs=pl.BlockSpec((_BM, h2), lambda i: (i, 0)),
        out_shape=jax.ShapeDtypeStruct((n, h2), jnp.float32),
        compiler_params=pltpu.CompilerParams(
            dimension_semantics=("arbitrary",),
        ),
    )(adj, s2, b2r)

    return out


# speedup vs baseline: 1.0609x; 1.0609x over previous
import jax
import jax.numpy as jnp
from jax.experimental import pallas as pl
from jax.experimental.pallas import tpu as pltpu

_BM = 400


def _gcn_body(x_ref, adj_ref, w1_ref, b1_ref, w2_ref, b2_ref,
              out_ref, sup_ref, s2_ref):
    p = pl.program_id(0)
    i = pl.program_id(1)

    @pl.when((p == 0) & (i == 0))
    def _():
        sup_ref[...] = jnp.dot(
            x_ref[...], w1_ref[...], preferred_element_type=jnp.float32)

    @pl.when(p == 0)
    def _():
        acc = jnp.dot(
            adj_ref[...], sup_ref[...], preferred_element_type=jnp.float32)
        h = jnp.maximum(acc + b1_ref[...], 0.0)
        s2_ref[pl.ds(i * _BM, _BM), :] = jnp.dot(
            h, w2_ref[...], preferred_element_type=jnp.float32)

    @pl.when(p == 1)
    def _():
        out_ref[...] = jnp.dot(
            adj_ref[...], s2_ref[...], preferred_element_type=jnp.float32
        ) + b2_ref[...]


def kernel(x, adj, W1, b1, W2, b2):
    n, nfeat = x.shape
    h1 = W1.shape[1]
    h2 = W2.shape[1]
    b1r = b1.reshape(1, h1)
    b2r = b2.reshape(1, h2)

    out = pl.pallas_call(
        _gcn_body,
        grid=(2, n // _BM),
        in_specs=[
            pl.BlockSpec((n, nfeat), lambda p, i: (0, 0)),
            pl.BlockSpec((_BM, n), lambda p, i: (i, 0)),
            pl.BlockSpec((nfeat, h1), lambda p, i: (0, 0)),
            pl.BlockSpec((1, h1), lambda p, i: (0, 0)),
            pl.BlockSpec((h1, h2), lambda p, i: (0, 0)),
            pl.BlockSpec((1, h2), lambda p, i: (0, 0)),
        ],
        out_specs=pl.BlockSpec((_BM, h2), lambda p, i: (p * i, 0)),
        out_shape=jax.ShapeDtypeStruct((n, h2), jnp.float32),
        scratch_shapes=[
            pltpu.VMEM((n, h1), jnp.float32),
            pltpu.VMEM((n, h2), jnp.float32),
        ],
        compiler_params=pltpu.CompilerParams(
            dimension_semantics=("arbitrary", "arbitrary"),
        ),
    )(x, adj, W1, b1r, W2, b2r)

    return out
